# baseline (speedup 1.0000x reference)
import functools

import jax
import jax.numpy as jnp
from jax import lax
from jax.experimental import pallas as pl
from jax.experimental.pallas import tpu as pltpu
from jax.experimental.pallas import tpu_sc as plsc

BATCH = 4096
N_GROUPS = 26
INNER = 64
OUT = 128
N_K = N_GROUPS // 2
A_ROWS = N_K * BATCH

NC = 2
NS = 16
NW = NC * NS
BB = BATCH // NW


def _gather_kmajor(x, table):
    mesh = plsc.VectorSubcoreMesh(core_axis_name="c", subcore_axis_name="s")

    @functools.partial(
        pl.kernel,
        out_type=jax.ShapeDtypeStruct((A_ROWS, 2 * INNER), jnp.float32),
        mesh=mesh,
        scratch_types=[
            pltpu.VMEM((BB, N_GROUPS), jnp.int32),
            pltpu.VMEM((BB,), jnp.int32),
            pltpu.VMEM((BB,), jnp.int32),
            pltpu.VMEM((BB, INNER), jnp.float32),
            pltpu.VMEM((BB, INNER), jnp.float32),
            pltpu.SemaphoreType.DMA,
            pltpu.SemaphoreType.DMA,
            pltpu.SemaphoreType.DMA,
            pltpu.SemaphoreType.DMA,
        ],
        compiler_params=pltpu.CompilerParams(use_tc_tiling_on_sc=False,
                                             needs_layout_passes=False),
    )
    def gather_kernel(x_hbm, table_hbm, out_hbm, idx2d, ic0, ic1, rb0, rb1,
                      gs0, gs1, os0, os1):
        wid = lax.axis_index("s") * NC + lax.axis_index("c")
        b0 = wid * BB
        pltpu.sync_copy(x_hbm.at[pl.ds(b0, BB)], idx2d)
        ics = (ic0, ic1)
        rbs = (rb0, rb1)
        gss = (gs0, gs1)
        oss = (os0, os1)

        def body(k, carry):
            for p in (0, 1):
                g = 2 * k + p
                icol, rbuf, gsem, osem = ics[p], rbs[p], gss[p], oss[p]
                gvec = jnp.full((16,), 0, jnp.int32) + g
                for i in range(BB // 16):
                    rows = lax.iota(jnp.int32, 16) + (16 * i)
                    icol[pl.ds(16 * i, 16)] = plsc.load_gather(
                        idx2d, [rows, gvec])
                dst = out_hbm.at[pl.ds(k * BATCH + b0, BB),
                                 pl.ds(p * INNER, INNER)]

                @pl.when(k > 0)
                def _():
                    pltpu.make_async_copy(rbuf, dst, osem).wait()

                pltpu.async_copy(table_hbm.at[icol], rbuf, gsem).wait()
                pltpu.async_copy(rbuf, dst, osem)
            return carry

        lax.fori_loop(0, N_K, body, 0)
        for p in (0, 1):
            dst = out_hbm.at[pl.ds((N_K - 1) * BATCH + b0, BB),
                             pl.ds(p * INNER, INNER)]
            pltpu.make_async_copy(rbs[p], dst, oss[p]).wait()

    return gather_kernel(x, table)


def _project_kmajor(a, w):
    bm = 512
    nb = BATCH // bm

    def mm(a_ref, w_ref, o_ref):
        k = pl.program_id(1)

        @pl.when(k == 0)
        def _():
            o_ref[...] = jnp.zeros_like(o_ref)

        o_ref[...] += lax.dot_general(
            a_ref[...], w_ref[...], (((1,), (1,)), ((), ())),
            preferred_element_type=jnp.float32)

    return pl.pallas_call(
        mm,
        grid=(nb, N_K),
        in_specs=[
            pl.BlockSpec((bm, 2 * INNER), lambda i, k: (k * nb + i, 0)),
            pl.BlockSpec((OUT, 2 * INNER), lambda i, k: (0, k)),
        ],
        out_specs=pl.BlockSpec((bm, OUT), lambda i, k: (i, 0)),
        out_shape=jax.ShapeDtypeStruct((BATCH, OUT), jnp.float32),
    )(a, w)


def kernel(x, table, W):
    a = _gather_kmajor(x.astype(jnp.int32), table)
    return _project_kmajor(a, W)


# speedup vs baseline: 1.3773x; 1.3773x over previous
import functools

import jax
import jax.numpy as jnp
from jax import lax
from jax.experimental import pallas as pl
from jax.experimental.pallas import tpu as pltpu
from jax.experimental.pallas import tpu_sc as plsc

BATCH = 4096
N_GROUPS = 26
INNER = 64
OUT = 128
N_K = N_GROUPS // 2
A_ROWS = N_K * BATCH

NC = 2
NS = 16
NW = NC * NS
BB = BATCH // NW


def _gather_kmajor(x, table):
    mesh = plsc.VectorSubcoreMesh(core_axis_name="c", subcore_axis_name="s")

    @functools.partial(
        pl.kernel,
        out_type=jax.ShapeDtypeStruct((A_ROWS, 2 * INNER), jnp.float32),
        mesh=mesh,
        scratch_types=[
            pltpu.VMEM((BB, N_GROUPS), jnp.int32),
            pltpu.VMEM((BB,), jnp.int32),
            pltpu.VMEM((BB,), jnp.int32),
            pltpu.VMEM((BB, INNER), jnp.float32),
            pltpu.VMEM((BB, INNER), jnp.float32),
            pltpu.SemaphoreType.DMA,
            pltpu.SemaphoreType.DMA,
            pltpu.SemaphoreType.DMA,
            pltpu.SemaphoreType.DMA,
        ],
        compiler_params=pltpu.CompilerParams(use_tc_tiling_on_sc=False,
                                             needs_layout_passes=False),
    )
    def gather_kernel(x_hbm, table_hbm, out_hbm, idx2d, ic0, ic1, rb0, rb1,
                      gs0, gs1, os0, os1):
        wid = lax.axis_index("s") * NC + lax.axis_index("c")
        b0 = wid * BB
        pltpu.sync_copy(x_hbm.at[pl.ds(b0, BB)], idx2d)
        ics = (ic0, ic1)
        rbs = (rb0, rb1)
        gss = (gs0, gs1)
        oss = (os0, os1)

        def body(k, carry):
            for p in (0, 1):
                g = 2 * k + p
                icol, rbuf, gsem, osem = ics[p], rbs[p], gss[p], oss[p]
                gvec = jnp.full((16,), 0, jnp.int32) + g
                for i in range(BB // 16):
                    rows = lax.iota(jnp.int32, 16) + (16 * i)
                    icol[pl.ds(16 * i, 16)] = plsc.load_gather(
                        idx2d, [rows, gvec])
                dst = out_hbm.at[pl.ds(k * BATCH + b0, BB),
                                 pl.ds(p * INNER, INNER)]

                @pl.when(k > 0)
                def _():
                    pltpu.make_async_copy(rbuf, dst, osem).wait()

                pltpu.async_copy(table_hbm.at[icol], rbuf, gsem).wait()
                pltpu.async_copy(rbuf, dst, osem)
            return carry

        lax.fori_loop(0, N_K, body, 0)
        for p in (0, 1):
            dst = out_hbm.at[pl.ds((N_K - 1) * BATCH + b0, BB),
                             pl.ds(p * INNER, INNER)]
            pltpu.make_async_copy(rbs[p], dst, oss[p]).wait()

    return gather_kernel(x, table)


def _project_kmajor(a, w):
    def mm(a_ref, w_ref, o_ref):
        k = pl.program_id(0)

        @pl.when(k == 0)
        def _():
            o_ref[...] = jnp.zeros_like(o_ref)

        o_ref[...] += lax.dot_general(
            a_ref[...], w_ref[...], (((1,), (1,)), ((), ())),
            preferred_element_type=jnp.float32)

    return pl.pallas_call(
        mm,
        grid=(N_K,),
        in_specs=[
            pl.BlockSpec((BATCH, 2 * INNER), lambda k: (k, 0)),
            pl.BlockSpec((OUT, 2 * INNER), lambda k: (0, k)),
        ],
        out_specs=pl.BlockSpec((BATCH, OUT), lambda k: (0, 0)),
        out_shape=jax.ShapeDtypeStruct((BATCH, OUT), jnp.float32),
    )(a, w)


def kernel(x, table, W):
    a = _gather_kmajor(x.astype(jnp.int32), table)
    return _project_kmajor(a, W)
